# baseline (speedup 1.0000x reference)
import jax
import jax.numpy as jnp
from jax.experimental import pallas as pl

_BL = 512
_CG = 2


def _merge(x_ref, o_ref):
    cg, h, w, bl = x_ref.shape
    o_ref[...] = x_ref[...].reshape(cg * h * w, bl)


def kernel(embedded_tasks):
    b, c, h, w = embedded_tasks.shape
    f = c * h * w
    xt = jnp.transpose(embedded_tasks, (1, 2, 3, 0))
    yt = pl.pallas_call(
        _merge,
        grid=(c // _CG, b // _BL),
        in_specs=[
            pl.BlockSpec((_CG, h, w, _BL), lambda i, j: (i, 0, 0, j))
        ],
        out_specs=pl.BlockSpec((_CG * h * w, _BL), lambda i, j: (i, j)),
        out_shape=jax.ShapeDtypeStruct((f, b), jnp.float32),
    )(xt)
    return yt.T


# speedup vs baseline: 1.0307x; 1.0307x over previous
import jax
import jax.numpy as jnp
from jax.experimental import pallas as pl

_BL = 1024
_CG = 4


def _merge(x_ref, o_ref):
    cg, h, w, bl = x_ref.shape
    o_ref[...] = x_ref[...].reshape(cg * h * w, bl)


def kernel(embedded_tasks):
    b, c, h, w = embedded_tasks.shape
    f = c * h * w
    xt = jnp.transpose(embedded_tasks, (1, 2, 3, 0))
    yt = pl.pallas_call(
        _merge,
        grid=(c // _CG, b // _BL),
        in_specs=[
            pl.BlockSpec((_CG, h, w, _BL), lambda i, j: (i, 0, 0, j))
        ],
        out_specs=pl.BlockSpec((_CG * h * w, _BL), lambda i, j: (i, j)),
        out_shape=jax.ShapeDtypeStruct((f, b), jnp.float32),
    )(xt)
    return yt.T


# speedup vs baseline: 1.0324x; 1.0016x over previous
import jax
import jax.numpy as jnp
from jax.experimental import pallas as pl

_BL = 512
_CG = 6


def _merge(x_ref, o_ref):
    cg, h, w, bl = x_ref.shape
    o_ref[...] = x_ref[...].reshape(cg * h * w, bl)


def kernel(embedded_tasks):
    b, c, h, w = embedded_tasks.shape
    f = c * h * w
    xt = jnp.transpose(embedded_tasks, (1, 2, 3, 0))
    yt = pl.pallas_call(
        _merge,
        grid=(c // _CG, b // _BL),
        in_specs=[
            pl.BlockSpec((_CG, h, w, _BL), lambda i, j: (i, 0, 0, j))
        ],
        out_specs=pl.BlockSpec((_CG * h * w, _BL), lambda i, j: (i, j)),
        out_shape=jax.ShapeDtypeStruct((f, b), jnp.float32),
    )(xt)
    return yt.T
